# baseline (speedup 1.0000x reference)
import functools

import jax
import jax.numpy as jnp
from jax import lax
from jax.experimental import pallas as pl
from jax.experimental.pallas import tpu as pltpu
from jax.experimental.pallas import tpu_sc as plsc

N_CLASSES = 1000000
EMBED_DIM = 64
BATCH = 16384

_INFO = plsc.get_sparse_core_info()
_NC = _INFO.num_cores
_NS = _INFO.num_subcores
_NW = _NC * _NS
_B_PER_W = BATCH // _NW
_CHUNK = 128
_NCHUNKS = _B_PER_W // _CHUNK

_BC = 32768
_GRID = (N_CLASSES + _BC - 1) // _BC


def _transpose_pad_body(tt_ref, out_ref):
    blk = tt_ref[...]
    out_ref[...] = jnp.pad(blk.T, ((0, 0), (0, EMBED_DIM)))


_transpose_pad = pl.pallas_call(
    _transpose_pad_body,
    grid=(_GRID,),
    in_specs=[pl.BlockSpec((EMBED_DIM, _BC), lambda i: (0, i))],
    out_specs=pl.BlockSpec((_BC, 2 * EMBED_DIM), lambda i: (i, 0)),
    out_shape=jax.ShapeDtypeStruct((N_CLASSES, 2 * EMBED_DIM), jnp.float32),
)


@functools.partial(
    pl.kernel,
    mesh=plsc.VectorSubcoreMesh(core_axis_name="c", subcore_axis_name="s"),
    out_type=jax.ShapeDtypeStruct((EMBED_DIM, BATCH), jnp.float32),
    scratch_types=[
        pltpu.VMEM((_NCHUNKS, _CHUNK), jnp.int32),
        pltpu.VMEM((_B_PER_W, 2 * EMBED_DIM), jnp.float32),
        pltpu.VMEM((EMBED_DIM, _B_PER_W), jnp.float32),
        pltpu.SemaphoreType.DMA,
    ],
    compiler_params=pltpu.CompilerParams(needs_layout_passes=False),
)
def _embed_lookup(idx_hbm, tablep_hbm, out_hbm, idx_v, rows_v, cols_v, sem):
    wid = lax.axis_index("s") * _NC + lax.axis_index("c")
    base = wid * _B_PER_W
    pltpu.sync_copy(idx_hbm.at[wid], idx_v)
    copies = []
    for j in range(_NCHUNKS):
        copies.append(
            pltpu.async_copy(
                tablep_hbm.at[idx_v.at[j]],
                rows_v.at[pl.ds(j * _CHUNK, _CHUNK)],
                sem,
            )
        )
    for c in copies:
        c.wait()
    lanes = lax.iota(jnp.int32, 16)

    def transpose_group(g):
        kvec = lanes + g * 16
        for f in range(EMBED_DIM):
            vals = plsc.load_gather(rows_v, [kvec, lanes * 0 + f])
            cols_v[f, pl.ds(g * 16, 16)] = vals

    pl.loop(0, _B_PER_W // 16)(transpose_group)
    pltpu.sync_copy(cols_v, out_hbm.at[:, pl.ds(base, _B_PER_W)])


def kernel(class_ids, table):
    idx = class_ids.astype(jnp.int32).reshape(_NW, _NCHUNKS, _CHUNK)
    table_p = _transpose_pad(table.T)
    out_t = _embed_lookup(idx, table_p)
    return out_t.T.reshape(BATCH, 1, EMBED_DIM)


# speedup vs baseline: 1.0353x; 1.0353x over previous
import functools

import jax
import jax.numpy as jnp
from jax import lax
from jax.experimental import pallas as pl
from jax.experimental.pallas import tpu as pltpu
from jax.experimental.pallas import tpu_sc as plsc

N_CLASSES = 1000000
EMBED_DIM = 64
BATCH = 16384

_INFO = plsc.get_sparse_core_info()
_NC = _INFO.num_cores
_NS = _INFO.num_subcores
_NW = _NC * _NS
_B_PER_W = BATCH // _NW
_CHUNK = 128
_NCHUNKS = _B_PER_W // _CHUNK

_BC = 32768
_GRID = (N_CLASSES + _BC - 1) // _BC


def _transpose_pad_body(tt_ref, out_ref):
    blk = tt_ref[...]
    out_ref[...] = jnp.pad(blk.T, ((0, 0), (0, EMBED_DIM)))


_transpose_pad = pl.pallas_call(
    _transpose_pad_body,
    grid=(_GRID,),
    in_specs=[pl.BlockSpec((EMBED_DIM, _BC), lambda i: (0, i))],
    out_specs=pl.BlockSpec((_BC, 2 * EMBED_DIM), lambda i: (i, 0)),
    out_shape=jax.ShapeDtypeStruct((N_CLASSES, 2 * EMBED_DIM), jnp.float32),
)


@functools.partial(
    pl.kernel,
    mesh=plsc.VectorSubcoreMesh(core_axis_name="c", subcore_axis_name="s"),
    out_type=jax.ShapeDtypeStruct((BATCH, 2 * EMBED_DIM), jnp.float32),
    scratch_types=[
        pltpu.VMEM((_NCHUNKS, _CHUNK), jnp.int32),
        pltpu.VMEM((_B_PER_W, 2 * EMBED_DIM), jnp.float32),
        pltpu.SemaphoreType.DMA,
    ],
)
def _embed_lookup(idx_hbm, tablep_hbm, out_hbm, idx_v, rows_v, sem):
    wid = lax.axis_index("s") * _NC + lax.axis_index("c")
    base = wid * _B_PER_W
    pltpu.sync_copy(idx_hbm.at[wid], idx_v)
    copies = []
    for j in range(_NCHUNKS):
        copies.append(
            pltpu.async_copy(
                tablep_hbm.at[idx_v.at[j]],
                rows_v.at[pl.ds(j * _CHUNK, _CHUNK)],
                sem,
            )
        )
    for c in copies:
        c.wait()
    pltpu.sync_copy(rows_v, out_hbm.at[pl.ds(base, _B_PER_W)])


def kernel(class_ids, table):
    idx = class_ids.astype(jnp.int32).reshape(_NW, _NCHUNKS, _CHUNK)
    table_p = _transpose_pad(table.T)
    out = _embed_lookup(idx, table_p)
    return out[:, :EMBED_DIM].reshape(BATCH, 1, EMBED_DIM)
